# baseline (speedup 1.0000x reference)
import dataclasses
import functools

import jax
import jax.numpy as jnp
from jax import lax
from jax.experimental import pallas as pl
from jax.experimental.pallas import tpu as pltpu
from jax.experimental.pallas import tpu_sc as plsc

N = 10000
D = 256
E = 160000

NC = 2
NS = 16
NW = NC * NS

DEG_PAD = 10240
E_PER_TILE_H = E // NW
H_FULL = E_PER_TILE_H // 16
H_TAIL = E_PER_TILE_H - H_FULL * 16
DEG_SL = DEG_PAD // NS

GRP = 8
ACC_ROWS = 320
CH_E = 3200
N_CH = E // CH_E
G_ROWS = 112

BLK = 1000


def _vmesh():
    return plsc.VectorSubcoreMesh(
        core_axis_name="c", subcore_axis_name="s",
        num_cores=NC, num_subcores=NS)


def _sc_params():
    cp = pltpu.CompilerParams()
    if "needs_layout_passes" in pltpu.CompilerParams.__dataclass_fields__:
        cp = dataclasses.replace(cp, needs_layout_passes=False)
    return cp


@functools.partial(
    pl.kernel,
    out_type=jax.ShapeDtypeStruct((NC * DEG_PAD,), jnp.float32),
    mesh=_vmesh(),
    compiler_params=_sc_params(),
    scratch_types=[
        pltpu.VMEM((E_PER_TILE_H + 8,), jnp.int32),
        pltpu.VMEM((DEG_PAD,), jnp.float32),
        pltpu.VMEM((NS, DEG_SL), jnp.float32),
        pltpu.VMEM((DEG_SL,), jnp.float32),
        pltpu.VMEM_SHARED((NS, DEG_PAD), jnp.float32),
    ],
)
def _deg_kernel(dst_hbm, zeros_hbm, degp_hbm,
                dst_v, hist_v, red_v, col_v, hist_sp):
    c = lax.axis_index("c")
    s = lax.axis_index("s")
    w = c * NS + s
    pltpu.sync_copy(zeros_hbm, hist_v)
    dst_v[pl.ds(H_FULL * 16, 16)] = jnp.full((16,), DEG_PAD - 1, jnp.int32)
    pltpu.sync_copy(dst_hbm.at[pl.ds(w * E_PER_TILE_H, E_PER_TILE_H)],
                    dst_v.at[pl.ds(0, E_PER_TILE_H)])
    ones16 = jnp.full((16,), 1.0, jnp.float32)
    lanes = lax.iota(jnp.int32, 16)

    @pl.loop(0, H_FULL + 1)
    def _(g):
        d = dst_v[pl.ds(g * 16, 16)]
        for j in range(16):
            plsc.addupdate_scatter(hist_v, [d], ones16, mask=lanes == j)

    pltpu.sync_copy(hist_v, hist_sp.at[s])
    plsc.subcore_barrier()
    pltpu.sync_copy(hist_sp.at[:, pl.ds(s * DEG_SL, DEG_SL)], red_v)

    @pl.loop(0, DEG_SL // 16)
    def _(g):
        acc = red_v[0, pl.ds(g * 16, 16)]
        for r in range(1, NS):
            acc = acc + red_v[r, pl.ds(g * 16, 16)]
        col_v[pl.ds(g * 16, 16)] = acc

    pltpu.sync_copy(col_v, degp_hbm.at[pl.ds(c * DEG_PAD + s * DEG_SL, DEG_SL)])


@functools.partial(
    pl.kernel,
    out_type=jax.ShapeDtypeStruct((N, D), jnp.float32),
    mesh=_vmesh(),
    compiler_params=_sc_params(),
    scratch_types=[
        pltpu.VMEM((CH_E + 16,), jnp.int32),
        pltpu.VMEM((CH_E + 16,), jnp.int32),
        pltpu.VMEM((G_ROWS, D), jnp.float32),
        pltpu.VMEM((ACC_ROWS, D), jnp.float32),
    ],
)
def _agg_kernel(y_hbm, src_hbm, dst_hbm, zrows_hbm, out_hbm,
                csrc_v, clid_v, rows_v, acc_v):
    c = lax.axis_index("c")
    s = lax.axis_index("s")
    w = c * NS + s
    lanes = lax.iota(jnp.int32, 16)

    pltpu.sync_copy(zrows_hbm, acc_v.at[pl.ds(0, G_ROWS)])
    pltpu.sync_copy(zrows_hbm, acc_v.at[pl.ds(G_ROWS, G_ROWS)])
    pltpu.sync_copy(zrows_hbm.at[pl.ds(0, ACC_ROWS - 2 * G_ROWS)],
                    acc_v.at[pl.ds(2 * G_ROWS, ACC_ROWS - 2 * G_ROWS)])

    def chunk_body(src_vm, dst_vm):
        def cbody(g2, off):
            g = g2 * 2
            for u in range(2):
                d = dst_vm[0, pl.ds((g + u) * 16, 16)]
                sg = src_vm[0, pl.ds((g + u) * 16, 16)]
                m = ((d >> 3) & 31) == w
                lidx = ((d >> 8) << 3) | (d & 7)
                plsc.store_compressed(csrc_v.at[pl.ds(off, 16)], sg, mask=m)
                plsc.store_compressed(clid_v.at[pl.ds(off, 16)], lidx, mask=m)
                off = off + jnp.sum(m.astype(jnp.int32))
            return off

        k = lax.fori_loop(0, CH_E // 32, cbody, 0)
        nb = (k + G_ROWS - 1) // G_ROWS

        def gbody(gb, _):
            b0 = pl.multiple_of(gb * G_ROWS, 8)
            pltpu.sync_copy(y_hbm.at[csrc_v.at[pl.ds(b0, G_ROWS)]], rows_v)
            cnt = jnp.minimum(G_ROWS, k - b0)

            def fgroup(q, _):
                lvec = clid_v[pl.ds(b0 + q * 16, 16)]
                rowi = q * 16 + lanes
                msk = lanes < (cnt - q * 16)
                for col in range(D):
                    cc = jnp.full((16,), col, jnp.int32)
                    v = plsc.load_gather(rows_v, [rowi, cc])
                    plsc.addupdate_scatter(acc_v, [lvec, cc], v, mask=msk)
                return 0

            lax.fori_loop(0, (cnt + 15) >> 4, fgroup, 0)
            return 0

        lax.fori_loop(0, nb, gbody, 0)

    pltpu.emit_pipeline(
        chunk_body,
        grid=(N_CH,),
        in_specs=[
            pl.BlockSpec((1, CH_E), lambda i: (i, 0)),
            pl.BlockSpec((1, CH_E), lambda i: (i, 0)),
        ],
    )(src_hbm, dst_hbm)

    nfull = jnp.where(w < 2, 40, 39)

    def obody(i, _):
        l0 = pl.multiple_of(i * GRP, 8)
        g0 = pl.multiple_of((i * NW + w) * GRP, 8)
        pltpu.sync_copy(acc_v.at[pl.ds(l0, GRP)], out_hbm.at[pl.ds(g0, GRP)])
        return 0

    lax.fori_loop(0, nfull, obody, 0)


def _prep_body(x_ref, w_ref, b_ref, d0_ref, d1_ref, y_ref, selfb_ref, dis_ref):
    xw = lax.dot_general(x_ref[...], w_ref[...], (((1,), (0,)), ((), ())),
                         preferred_element_type=jnp.float32,
                         precision=lax.Precision.HIGHEST)
    deg = d0_ref[...] + d1_ref[...] + 1.0
    dinv = 1.0 / deg
    dis = jnp.sqrt(dinv)
    y_ref[...] = xw * dis
    selfb_ref[...] = xw * dinv + b_ref[...]
    dis_ref[...] = dis


def _combine_body(agg_ref, dis_ref, selfb_ref, o_ref):
    o_ref[...] = agg_ref[...] * dis_ref[...] + selfb_ref[...]


def kernel(x, edge_index, W, b):
    src = edge_index[0].astype(jnp.int32)
    dst = edge_index[1].astype(jnp.int32)
    zeros_d = jnp.zeros((DEG_PAD,), jnp.float32)
    zrows = jnp.zeros((G_ROWS, D), jnp.float32)

    degp = _deg_kernel(dst, zeros_d)
    d0 = degp[:N][:, None]
    d1 = degp[DEG_PAD:DEG_PAD + N][:, None]

    y, selfb, dis_col = pl.pallas_call(
        _prep_body,
        grid=(N // BLK,),
        in_specs=[
            pl.BlockSpec((BLK, D), lambda i: (i, 0)),
            pl.BlockSpec((D, D), lambda i: (0, 0)),
            pl.BlockSpec((1, D), lambda i: (0, 0)),
            pl.BlockSpec((BLK, 1), lambda i: (i, 0)),
            pl.BlockSpec((BLK, 1), lambda i: (i, 0)),
        ],
        out_specs=[
            pl.BlockSpec((BLK, D), lambda i: (i, 0)),
            pl.BlockSpec((BLK, D), lambda i: (i, 0)),
            pl.BlockSpec((BLK, 1), lambda i: (i, 0)),
        ],
        out_shape=[
            jax.ShapeDtypeStruct((N, D), jnp.float32),
            jax.ShapeDtypeStruct((N, D), jnp.float32),
            jax.ShapeDtypeStruct((N, 1), jnp.float32),
        ],
    )(x, W, b[None, :], d0, d1)

    agg = _agg_kernel(y, src.reshape(N_CH, CH_E), dst.reshape(N_CH, CH_E),
                      zrows)

    out = pl.pallas_call(
        _combine_body,
        grid=(N // BLK,),
        in_specs=[
            pl.BlockSpec((BLK, D), lambda i: (i, 0)),
            pl.BlockSpec((BLK, 1), lambda i: (i, 0)),
            pl.BlockSpec((BLK, D), lambda i: (i, 0)),
        ],
        out_specs=pl.BlockSpec((BLK, D), lambda i: (i, 0)),
        out_shape=jax.ShapeDtypeStruct((N, D), jnp.float32),
    )(agg, dis_col, selfb)
    return out


# speedup vs baseline: 2.9269x; 2.9269x over previous
import dataclasses
import functools

import jax
import jax.numpy as jnp
from jax import lax
from jax.experimental import pallas as pl
from jax.experimental.pallas import tpu as pltpu
from jax.experimental.pallas import tpu_sc as plsc

N = 10000
D = 256
E = 160000

NC = 2
NS = 16
NW = NC * NS

DEG_PAD = 10240
E_PER_TILE_H = E // NW
H_FULL = E_PER_TILE_H // 16
H_TAIL = E_PER_TILE_H - H_FULL * 16
DEG_SL = DEG_PAD // NS

GRP = 8
ACC_ROWS = 320
CH_E = 3200
N_CH = E // CH_E
G_ROWS = 96

BLK = 1000


def _vmesh():
    return plsc.VectorSubcoreMesh(
        core_axis_name="c", subcore_axis_name="s",
        num_cores=NC, num_subcores=NS)


def _sc_params():
    cp = pltpu.CompilerParams()
    if "needs_layout_passes" in pltpu.CompilerParams.__dataclass_fields__:
        cp = dataclasses.replace(cp, needs_layout_passes=False)
    return cp


@functools.partial(
    pl.kernel,
    out_type=jax.ShapeDtypeStruct((NC * DEG_PAD,), jnp.float32),
    mesh=_vmesh(),
    compiler_params=_sc_params(),
    scratch_types=[
        pltpu.VMEM((E_PER_TILE_H + 8,), jnp.int32),
        pltpu.VMEM((DEG_PAD,), jnp.float32),
        pltpu.VMEM((NS, DEG_SL), jnp.float32),
        pltpu.VMEM((DEG_SL,), jnp.float32),
        pltpu.VMEM_SHARED((NS, DEG_PAD), jnp.float32),
    ],
)
def _deg_kernel(dst_hbm, zeros_hbm, degp_hbm,
                dst_v, hist_v, red_v, col_v, hist_sp):
    c = lax.axis_index("c")
    s = lax.axis_index("s")
    w = c * NS + s
    pltpu.sync_copy(zeros_hbm, hist_v)
    dst_v[pl.ds(H_FULL * 16, 16)] = jnp.full((16,), DEG_PAD - 1, jnp.int32)
    pltpu.sync_copy(dst_hbm.at[pl.ds(w * E_PER_TILE_H, E_PER_TILE_H)],
                    dst_v.at[pl.ds(0, E_PER_TILE_H)])
    ones16 = jnp.full((16,), 1.0, jnp.float32)
    lanes = lax.iota(jnp.int32, 16)

    @pl.loop(0, H_FULL + 1)
    def _(g):
        d = dst_v[pl.ds(g * 16, 16)]
        for j in range(16):
            plsc.addupdate_scatter(hist_v, [d], ones16, mask=lanes == j)

    pltpu.sync_copy(hist_v, hist_sp.at[s])
    plsc.subcore_barrier()
    pltpu.sync_copy(hist_sp.at[:, pl.ds(s * DEG_SL, DEG_SL)], red_v)

    @pl.loop(0, DEG_SL // 16)
    def _(g):
        acc = red_v[0, pl.ds(g * 16, 16)]
        for r in range(1, NS):
            acc = acc + red_v[r, pl.ds(g * 16, 16)]
        col_v[pl.ds(g * 16, 16)] = acc

    pltpu.sync_copy(col_v, degp_hbm.at[pl.ds(c * DEG_PAD + s * DEG_SL, DEG_SL)])


@functools.partial(
    pl.kernel,
    out_type=jax.ShapeDtypeStruct((N, D), jnp.float32),
    mesh=_vmesh(),
    compiler_params=_sc_params(),
    scratch_types=[
        pltpu.VMEM((CH_E + 16,), jnp.int32),
        pltpu.VMEM((CH_E + 16,), jnp.int32),
        pltpu.VMEM((G_ROWS, D), jnp.float32),
        pltpu.VMEM((ACC_ROWS, D), jnp.float32),
    ],
)
def _agg_kernel(y_hbm, src_hbm, dst_hbm, zrows_hbm, out_hbm,
                csrc_v, clid_v, rows_v, acc_v):
    c = lax.axis_index("c")
    s = lax.axis_index("s")
    w = c * NS + s
    lanes = lax.iota(jnp.int32, 16)

    pltpu.sync_copy(zrows_hbm, acc_v.at[pl.ds(0, G_ROWS)])
    pltpu.sync_copy(zrows_hbm, acc_v.at[pl.ds(G_ROWS, G_ROWS)])
    pltpu.sync_copy(zrows_hbm, acc_v.at[pl.ds(2 * G_ROWS, G_ROWS)])
    pltpu.sync_copy(zrows_hbm.at[pl.ds(0, ACC_ROWS - 3 * G_ROWS)],
                    acc_v.at[pl.ds(3 * G_ROWS, ACC_ROWS - 3 * G_ROWS)])

    def chunk_body(src_vm, dst_vm):
        def cbody(g2, off):
            g = g2 * 2
            for u in range(2):
                d = dst_vm[0, pl.ds((g + u) * 16, 16)]
                sg = src_vm[0, pl.ds((g + u) * 16, 16)]
                m = ((d >> 3) & 31) == w
                lidx = ((d >> 8) << 3) | (d & 7)
                plsc.store_compressed(csrc_v.at[pl.ds(off, 16)], sg, mask=m)
                plsc.store_compressed(clid_v.at[pl.ds(off, 16)], lidx, mask=m)
                off = off + jnp.sum(m.astype(jnp.int32))
            return off

        k = lax.fori_loop(0, CH_E // 32, cbody, 0)
        nb = (k + G_ROWS - 1) // G_ROWS

        def gbody(gb, _):
            b0 = pl.multiple_of(gb * G_ROWS, 8)
            pltpu.sync_copy(y_hbm.at[csrc_v.at[pl.ds(b0, G_ROWS)]], rows_v)
            cnt = jnp.minimum(G_ROWS, k - b0)

            def fgroup(q, _):
                lvec = clid_v[pl.ds(b0 + q * 16, 16)]
                rowi = q * 16 + lanes
                msk = lanes < (cnt - q * 16)
                @pl.loop(0, D // 16)
                def _(c0):
                    base = c0 * 16
                    for i in range(16):
                        cc = (lanes + (base + i)) & (D - 1)
                        v = plsc.load_gather(rows_v, [rowi, cc])
                        plsc.addupdate_scatter(acc_v, [lvec, cc], v,
                                               mask=msk)
                return 0

            lax.fori_loop(0, (cnt + 15) >> 4, fgroup, 0)
            return 0

        lax.fori_loop(0, nb, gbody, 0)

    pltpu.emit_pipeline(
        chunk_body,
        grid=(N_CH,),
        in_specs=[
            pl.BlockSpec((1, CH_E), lambda i: (i, 0)),
            pl.BlockSpec((1, CH_E), lambda i: (i, 0)),
        ],
    )(src_hbm, dst_hbm)

    nfull = jnp.where(w < 2, 40, 39)

    def obody(i, _):
        l0 = pl.multiple_of(i * GRP, 8)
        g0 = pl.multiple_of((i * NW + w) * GRP, 8)
        pltpu.sync_copy(acc_v.at[pl.ds(l0, GRP)], out_hbm.at[pl.ds(g0, GRP)])
        return 0

    lax.fori_loop(0, nfull, obody, 0)


def _prep_body(x_ref, w_ref, b_ref, d0_ref, d1_ref, y_ref, selfb_ref, dis_ref):
    xw = lax.dot_general(x_ref[...], w_ref[...], (((1,), (0,)), ((), ())),
                         preferred_element_type=jnp.float32,
                         precision=lax.Precision.HIGHEST)
    deg = d0_ref[...] + d1_ref[...] + 1.0
    dinv = 1.0 / deg
    dis = jnp.sqrt(dinv)
    y_ref[...] = xw * dis
    selfb_ref[...] = xw * dinv + b_ref[...]
    dis_ref[...] = dis


def _combine_body(agg_ref, dis_ref, selfb_ref, o_ref):
    o_ref[...] = agg_ref[...] * dis_ref[...] + selfb_ref[...]


def kernel(x, edge_index, W, b):
    src = edge_index[0].astype(jnp.int32)
    dst = edge_index[1].astype(jnp.int32)
    zeros_d = jnp.zeros((DEG_PAD,), jnp.float32)
    zrows = jnp.zeros((G_ROWS, D), jnp.float32)

    degp = _deg_kernel(dst, zeros_d)
    d0 = degp[:N][:, None]
    d1 = degp[DEG_PAD:DEG_PAD + N][:, None]

    y, selfb, dis_col = pl.pallas_call(
        _prep_body,
        grid=(N // BLK,),
        in_specs=[
            pl.BlockSpec((BLK, D), lambda i: (i, 0)),
            pl.BlockSpec((D, D), lambda i: (0, 0)),
            pl.BlockSpec((1, D), lambda i: (0, 0)),
            pl.BlockSpec((BLK, 1), lambda i: (i, 0)),
            pl.BlockSpec((BLK, 1), lambda i: (i, 0)),
        ],
        out_specs=[
            pl.BlockSpec((BLK, D), lambda i: (i, 0)),
            pl.BlockSpec((BLK, D), lambda i: (i, 0)),
            pl.BlockSpec((BLK, 1), lambda i: (i, 0)),
        ],
        out_shape=[
            jax.ShapeDtypeStruct((N, D), jnp.float32),
            jax.ShapeDtypeStruct((N, D), jnp.float32),
            jax.ShapeDtypeStruct((N, 1), jnp.float32),
        ],
    )(x, W, b[None, :], d0, d1)

    agg = _agg_kernel(y, src.reshape(N_CH, CH_E), dst.reshape(N_CH, CH_E),
                      zrows)

    out = pl.pallas_call(
        _combine_body,
        grid=(N // BLK,),
        in_specs=[
            pl.BlockSpec((BLK, D), lambda i: (i, 0)),
            pl.BlockSpec((BLK, 1), lambda i: (i, 0)),
            pl.BlockSpec((BLK, D), lambda i: (i, 0)),
        ],
        out_specs=pl.BlockSpec((BLK, D), lambda i: (i, 0)),
        out_shape=jax.ShapeDtypeStruct((N, D), jnp.float32),
    )(agg, dis_col, selfb)
    return out


# speedup vs baseline: 3.3544x; 1.1461x over previous
import dataclasses
import functools

import jax
import jax.numpy as jnp
from jax import lax
from jax.experimental import pallas as pl
from jax.experimental.pallas import tpu as pltpu
from jax.experimental.pallas import tpu_sc as plsc

N = 10000
D = 256
E = 160000

NC = 2
NS = 16
NW = NC * NS

DEG_PAD = 10240
E_PER_TILE_H = E // NW
H_FULL = E_PER_TILE_H // 16
H_TAIL = E_PER_TILE_H - H_FULL * 16
DEG_SL = DEG_PAD // NS

GRP = 8
ACC_ROWS = 320
CH_E = 3200
N_CH = E // CH_E
G_ROWS = 112

BLK = 1000


def _vmesh():
    return plsc.VectorSubcoreMesh(
        core_axis_name="c", subcore_axis_name="s",
        num_cores=NC, num_subcores=NS)


def _sc_params():
    cp = pltpu.CompilerParams()
    if "needs_layout_passes" in pltpu.CompilerParams.__dataclass_fields__:
        cp = dataclasses.replace(cp, needs_layout_passes=False)
    return cp


@functools.partial(
    pl.kernel,
    out_type=jax.ShapeDtypeStruct((NC * DEG_PAD,), jnp.float32),
    mesh=_vmesh(),
    compiler_params=_sc_params(),
    scratch_types=[
        pltpu.VMEM((E_PER_TILE_H + 8,), jnp.int32),
        pltpu.VMEM((DEG_PAD,), jnp.float32),
        pltpu.VMEM((NS, DEG_SL), jnp.float32),
        pltpu.VMEM((DEG_SL,), jnp.float32),
        pltpu.VMEM_SHARED((NS, DEG_PAD), jnp.float32),
    ],
)
def _deg_kernel(dst_hbm, zeros_hbm, degp_hbm,
                dst_v, hist_v, red_v, col_v, hist_sp):
    c = lax.axis_index("c")
    s = lax.axis_index("s")
    w = c * NS + s
    pltpu.sync_copy(zeros_hbm, hist_v)
    dst_v[pl.ds(H_FULL * 16, 16)] = jnp.full((16,), DEG_PAD - 1, jnp.int32)
    pltpu.sync_copy(dst_hbm.at[pl.ds(w * E_PER_TILE_H, E_PER_TILE_H)],
                    dst_v.at[pl.ds(0, E_PER_TILE_H)])
    ones16 = jnp.full((16,), 1.0, jnp.float32)
    lanes = lax.iota(jnp.int32, 16)

    @pl.loop(0, H_FULL + 1)
    def _(g):
        d = dst_v[pl.ds(g * 16, 16)]
        for j in range(16):
            plsc.addupdate_scatter(hist_v, [d], ones16, mask=lanes == j)

    pltpu.sync_copy(hist_v, hist_sp.at[s])
    plsc.subcore_barrier()
    pltpu.sync_copy(hist_sp.at[:, pl.ds(s * DEG_SL, DEG_SL)], red_v)

    @pl.loop(0, DEG_SL // 16)
    def _(g):
        acc = red_v[0, pl.ds(g * 16, 16)]
        for r in range(1, NS):
            acc = acc + red_v[r, pl.ds(g * 16, 16)]
        col_v[pl.ds(g * 16, 16)] = acc

    pltpu.sync_copy(col_v, degp_hbm.at[pl.ds(c * DEG_PAD + s * DEG_SL, DEG_SL)])


@functools.partial(
    pl.kernel,
    out_type=jax.ShapeDtypeStruct((N, D), jnp.float32),
    mesh=_vmesh(),
    compiler_params=_sc_params(),
    scratch_types=[
        pltpu.VMEM((CH_E + 16,), jnp.int32),
        pltpu.VMEM((CH_E + 16,), jnp.int32),
        pltpu.VMEM((G_ROWS, D), jnp.float32),
        pltpu.VMEM((ACC_ROWS, D), jnp.float32),
    ],
)
def _agg_kernel(y_hbm, src_hbm, dst_hbm, zrows_hbm, out_hbm,
                csrc_v, clid_v, rows_v, acc_v):
    c = lax.axis_index("c")
    s = lax.axis_index("s")
    w = c * NS + s
    lanes = lax.iota(jnp.int32, 16)

    pltpu.sync_copy(zrows_hbm, acc_v.at[pl.ds(0, G_ROWS)])
    pltpu.sync_copy(zrows_hbm, acc_v.at[pl.ds(G_ROWS, G_ROWS)])
    pltpu.sync_copy(zrows_hbm.at[pl.ds(0, ACC_ROWS - 2 * G_ROWS)],
                    acc_v.at[pl.ds(2 * G_ROWS, ACC_ROWS - 2 * G_ROWS)])

    def chunk_body(src_vm, dst_vm):
        def cbody(g2, off):
            g = g2 * 2
            for u in range(2):
                d = dst_vm[0, pl.ds((g + u) * 16, 16)]
                sg = src_vm[0, pl.ds((g + u) * 16, 16)]
                m = ((d >> 3) & 31) == w
                lidx = ((d >> 8) << 3) | (d & 7)
                plsc.store_compressed(csrc_v.at[pl.ds(off, 16)], sg, mask=m)
                plsc.store_compressed(clid_v.at[pl.ds(off, 16)], lidx, mask=m)
                off = off + jnp.sum(m.astype(jnp.int32))
            return off

        k = lax.fori_loop(0, CH_E // 32, cbody, 0)
        nb = (k + G_ROWS - 1) // G_ROWS

        def addrow(rr, li):
            for q in range(D // 16):
                plsc.addupdate(acc_v.at[li, pl.ds(q * 16, 16)],
                               rows_v[rr, pl.ds(q * 16, 16)])

        def gbody(gb, _):
            b0 = pl.multiple_of(gb * G_ROWS, 8)
            pltpu.sync_copy(y_hbm.at[csrc_v.at[pl.ds(b0, G_ROWS)]], rows_v)
            cnt = jnp.minimum(G_ROWS, k - b0)
            full = cnt >> 4

            def fgroup(q, _):
                lvec = clid_v[pl.ds(b0 + q * 16, 16)]
                lis = [jnp.sum(jnp.where(lanes == jj, lvec, 0))
                       for jj in range(16)]
                for jj in range(16):
                    addrow((q << 4) + jj, lis[jj])
                return 0

            lax.fori_loop(0, full, fgroup, 0)

            def rbody(r, _):
                qq = (r >> 4) << 4
                lvec = clid_v[pl.ds(b0 + qq, 16)]
                li = jnp.sum(jnp.where(lanes == (r - qq), lvec, 0))
                addrow(r, li)
                return 0

            lax.fori_loop(full << 4, cnt, rbody, 0)
            return 0

        lax.fori_loop(0, nb, gbody, 0)

    pltpu.emit_pipeline(
        chunk_body,
        grid=(N_CH,),
        in_specs=[
            pl.BlockSpec((1, CH_E), lambda i: (i, 0)),
            pl.BlockSpec((1, CH_E), lambda i: (i, 0)),
        ],
    )(src_hbm, dst_hbm)

    nfull = jnp.where(w < 2, 40, 39)

    def obody(i, _):
        l0 = pl.multiple_of(i * GRP, 8)
        g0 = pl.multiple_of((i * NW + w) * GRP, 8)
        pltpu.sync_copy(acc_v.at[pl.ds(l0, GRP)], out_hbm.at[pl.ds(g0, GRP)])
        return 0

    lax.fori_loop(0, nfull, obody, 0)


def _prep_body(x_ref, w_ref, b_ref, d0_ref, d1_ref, y_ref, selfb_ref, dis_ref):
    xw = lax.dot_general(x_ref[...], w_ref[...], (((1,), (0,)), ((), ())),
                         preferred_element_type=jnp.float32,
                         precision=lax.Precision.HIGHEST)
    deg = d0_ref[...] + d1_ref[...] + 1.0
    dinv = 1.0 / deg
    dis = jnp.sqrt(dinv)
    y_ref[...] = xw * dis
    selfb_ref[...] = xw * dinv + b_ref[...]
    dis_ref[...] = dis


def _combine_body(agg_ref, dis_ref, selfb_ref, o_ref):
    o_ref[...] = agg_ref[...] * dis_ref[...] + selfb_ref[...]


def kernel(x, edge_index, W, b):
    src = edge_index[0].astype(jnp.int32)
    dst = edge_index[1].astype(jnp.int32)
    zeros_d = jnp.zeros((DEG_PAD,), jnp.float32)
    zrows = jnp.zeros((G_ROWS, D), jnp.float32)

    degp = _deg_kernel(dst, zeros_d)
    d0 = degp[:N][:, None]
    d1 = degp[DEG_PAD:DEG_PAD + N][:, None]

    y, selfb, dis_col = pl.pallas_call(
        _prep_body,
        grid=(N // BLK,),
        in_specs=[
            pl.BlockSpec((BLK, D), lambda i: (i, 0)),
            pl.BlockSpec((D, D), lambda i: (0, 0)),
            pl.BlockSpec((1, D), lambda i: (0, 0)),
            pl.BlockSpec((BLK, 1), lambda i: (i, 0)),
            pl.BlockSpec((BLK, 1), lambda i: (i, 0)),
        ],
        out_specs=[
            pl.BlockSpec((BLK, D), lambda i: (i, 0)),
            pl.BlockSpec((BLK, D), lambda i: (i, 0)),
            pl.BlockSpec((BLK, 1), lambda i: (i, 0)),
        ],
        out_shape=[
            jax.ShapeDtypeStruct((N, D), jnp.float32),
            jax.ShapeDtypeStruct((N, D), jnp.float32),
            jax.ShapeDtypeStruct((N, 1), jnp.float32),
        ],
    )(x, W, b[None, :], d0, d1)

    agg = _agg_kernel(y, src.reshape(N_CH, CH_E), dst.reshape(N_CH, CH_E),
                      zrows)

    out = pl.pallas_call(
        _combine_body,
        grid=(N // BLK,),
        in_specs=[
            pl.BlockSpec((BLK, D), lambda i: (i, 0)),
            pl.BlockSpec((BLK, 1), lambda i: (i, 0)),
            pl.BlockSpec((BLK, D), lambda i: (i, 0)),
        ],
        out_specs=pl.BlockSpec((BLK, D), lambda i: (i, 0)),
        out_shape=jax.ShapeDtypeStruct((N, D), jnp.float32),
    )(agg, dis_col, selfb)
    return out
